# baseline (speedup 1.0000x reference)
import functools
import jax
import jax.numpy as jnp
from jax import lax
from jax.experimental import pallas as pl
from jax.experimental.pallas import tpu as pltpu
from jax.experimental.pallas import tpu_sc as plsc

_B = 4
_C = 96
_H = 224
_W = 224
_N = _H * _W
_P = 8
_BLK = 28
_K = 64
_KP = 32
_TN = 6272
_NT = _N // _TN
_BPAD = 896
_NSUB = 32
_TPS = (_B * _K) // _NSUB


def _im2col(x):
    xp = jnp.pad(x, ((0, 0), (0, 0), (1, 1), (1, 1)))
    sl = [xp[:, :, dh:dh + _H, dw:dw + _W] for dh in range(3) for dw in range(3)]
    col = jnp.stack(sl, axis=-1)
    col = col.transpose(0, 2, 3, 1, 4).reshape(_B, _N, 27)
    return jnp.pad(col, ((0, 0), (0, 0), (0, _KP - 27)))


def _resp_body(col_ref, wt_ref, b_ref, out_ref):
    x = col_ref[0]
    f = jnp.dot(x, wt_ref[...], preferred_element_type=jnp.float32)
    f = jnp.maximum(f + b_ref[...], 0.0)
    out_ref[0] = jnp.sum(f, axis=1, keepdims=True)


def _resp(colA, WfT, brow):
    return pl.pallas_call(
        _resp_body,
        grid=(_B, _NT),
        in_specs=[
            pl.BlockSpec((1, _TN, _KP), lambda b, n: (b, n, 0)),
            pl.BlockSpec((_KP, _C), lambda b, n: (0, 0)),
            pl.BlockSpec((1, _C), lambda b, n: (0, 0)),
        ],
        out_specs=pl.BlockSpec((1, _TN, 1), lambda b, n: (b, n, 0)),
        out_shape=jax.ShapeDtypeStruct((_B, _N, 1), jnp.float32),
    )(colA, WfT, brow)


def _lane_gather(x, idx):
    dn = lax.GatherDimensionNumbers(
        offset_dims=(), collapsed_slice_dims=(0,), start_index_map=(0,))
    return lax.gather(x, idx[:, None], dn, slice_sizes=(1,),
                      mode=lax.GatherScatterMode.PROMISE_IN_BOUNDS)


def _sel_body(resp_hbm, col_hbm, out_hbm, buf, rows, outv, sem):
    wid = lax.axis_index("s") * 2 + lax.axis_index("c")
    lanes = lax.iota(jnp.int32, 16)
    idxvec = jnp.zeros((16,), jnp.int32)
    subs = []
    rcs = []
    for j in range(_TPS):
        t = wid * _TPS + j
        pltpu.sync_copy(resp_hbm.at[t], buf)

        def body(i, carry):
            lb, li = carry
            v = buf[pl.ds(i * 16, 16)]
            upd = v > lb
            return jnp.where(upd, v, lb), jnp.where(upd, i, li)

        lb, li = lax.fori_loop(0, _BPAD // 16, body,
                               (jnp.full((16,), -jnp.inf, jnp.float32),
                                jnp.zeros((16,), jnp.int32)))
        cv = lb
        ci = li * 16 + lanes
        for s in (8, 4, 2, 1):
            perm = (lanes + s) & 15
            ov = _lane_gather(cv, perm)
            oi = _lane_gather(ci, perm)
            take = (ov > cv) | ((ov == cv) & (oi < ci))
            cv = jnp.where(take, ov, cv)
            ci = jnp.where(take, oi, ci)
        k = t & (_K - 1)
        b = t >> 6
        q = ((ci.astype(jnp.float32) + 0.5) * (1.0 / _BLK)).astype(jnp.int32)
        r = ci - q * _BLK
        row = (k >> 3) * _BLK + q
        col = (k & (_P - 1)) * _BLK + r
        gidx = b * _N + row * _W + col
        idxvec = jnp.where(lanes == j, gidx >> 2, idxvec)
        subs.append((gidx & 3) * _KP)
        rcs.append(jnp.where(lanes == 0, row.astype(jnp.float32),
                             jnp.where(lanes == 1, col.astype(jnp.float32),
                                       0.0)))
    pltpu.async_copy(col_hbm.at[idxvec], rows, sem).wait()
    zero = jnp.zeros((16,), jnp.float32)
    for j in range(_TPS):
        jv = jnp.full((16,), j, jnp.int32)
        v0 = plsc.load_gather(rows, [jv, subs[j] + lanes])
        v1 = plsc.load_gather(rows, [jv, subs[j] + 16 + lanes])
        outv[j, pl.ds(0, 16)] = v0
        outv[j, pl.ds(16, 16)] = v1
        outv[j, pl.ds(32, 16)] = rcs[j]
        for c in range(3, 8):
            outv[j, pl.ds(c * 16, 16)] = zero
    pltpu.sync_copy(outv, out_hbm.at[pl.ds(wid * _TPS, _TPS)])


def _select_gather(resp_blk, col128):
    mesh = plsc.VectorSubcoreMesh(core_axis_name="c", subcore_axis_name="s")
    f = pl.kernel(
        _sel_body,
        mesh=mesh,
        out_type=jax.ShapeDtypeStruct((_B * _K, 128), jnp.float32),
        scratch_types=[
            pltpu.VMEM((_BPAD,), jnp.float32),
            pltpu.VMEM((16, 128), jnp.float32),
            pltpu.VMEM((_TPS, 128), jnp.float32),
            pltpu.SemaphoreType.DMA,
        ],
        compiler_params=pltpu.CompilerParams(
            use_tc_tiling_on_sc=True, needs_layout_passes=False),
    )
    return f(resp_blk, col128)


def _dist_body(col_ref, pT_ref, rA_ref, cA_ref, wt_ref, w96_ref, brow_ref,
               bcol_ref, dr_ref, dc_ref, mv_ref, descT, nA, rmin, ridx):
    nt = pl.program_id(1)

    @pl.when(nt == 0)
    def _():
        d = jnp.dot(w96_ref[...], pT_ref[0], preferred_element_type=jnp.float32)
        d = jnp.maximum(d + bcol_ref[...], 0.0)
        descT[...] = d
        nA[...] = jnp.sum(d * d, axis=0, keepdims=True)
        rmin[...] = jnp.full((1, _K), jnp.inf, jnp.float32)
        ridx[...] = jnp.zeros((1, _K), jnp.int32)

    x = col_ref[0]
    f = jnp.dot(x, wt_ref[...], preferred_element_type=jnp.float32)
    f = jnp.maximum(f + brow_ref[...], 0.0)
    dots = jnp.dot(f, descT[...], preferred_element_type=jnp.float32)
    nb = jnp.sum(f * f, axis=1, keepdims=True)
    dist = nb - 2.0 * dots
    tmin = jnp.min(dist, axis=0, keepdims=True)
    ii = lax.broadcasted_iota(jnp.int32, (_TN, _K), 0)
    targ = jnp.min(jnp.where(dist == tmin, ii, jnp.int32(_TN)),
                   axis=0, keepdims=True)
    better = tmin < rmin[...]
    ridx[...] = jnp.where(better, targ + nt * _TN, ridx[...])
    rmin[...] = jnp.where(better, tmin, rmin[...])

    @pl.when(nt == _NT - 1)
    def _():
        idx = ridx[...]
        rB = (idx // _W).astype(jnp.float32)
        cB = (idx % _W).astype(jnp.float32)
        dr_ref[0] = rA_ref[0] - rB
        dc_ref[0] = cA_ref[0] - cB
        mv_ref[0] = rmin[...] + nA[...]


def _dist(colB, pT, rA, cA, WfT, W96, brow, bcol):
    out3 = [jax.ShapeDtypeStruct((_B, 1, _K), jnp.float32)] * 3
    return pl.pallas_call(
        _dist_body,
        grid=(_B, _NT),
        in_specs=[
            pl.BlockSpec((1, _TN, _KP), lambda b, n: (b, n, 0)),
            pl.BlockSpec((1, _KP, _K), lambda b, n: (b, 0, 0)),
            pl.BlockSpec((1, 1, _K), lambda b, n: (b, 0, 0)),
            pl.BlockSpec((1, 1, _K), lambda b, n: (b, 0, 0)),
            pl.BlockSpec((_KP, _C), lambda b, n: (0, 0)),
            pl.BlockSpec((_C, _KP), lambda b, n: (0, 0)),
            pl.BlockSpec((1, _C), lambda b, n: (0, 0)),
            pl.BlockSpec((_C, 1), lambda b, n: (0, 0)),
        ],
        out_specs=[pl.BlockSpec((1, 1, _K), lambda b, n: (b, 0, 0))] * 3,
        out_shape=out3,
        scratch_shapes=[
            pltpu.VMEM((_C, _K), jnp.float32),
            pltpu.VMEM((1, _K), jnp.float32),
            pltpu.VMEM((1, _K), jnp.float32),
            pltpu.VMEM((1, _K), jnp.int32),
        ],
        compiler_params=pltpu.CompilerParams(
            dimension_semantics=("arbitrary", "arbitrary")),
    )(colB, pT, rA, cA, WfT, W96, brow, bcol)


def _block_resp(resp):
    r = resp.reshape(_B, _P, _BLK, _P, _BLK)
    r = r.transpose(0, 1, 3, 2, 4).reshape(_B * _K, _BLK * _BLK)
    return jnp.pad(r, ((0, 0), (0, _BPAD - _BLK * _BLK)),
                   constant_values=-jnp.inf)


@jax.jit
def kernel(xA, xB, Wc, bc):
    colA = _im2col(xA)
    colB = _im2col(xB)
    Wf = Wc.reshape(_C, 27)
    W96 = jnp.pad(Wf, ((0, 0), (0, _KP - 27)))
    WfT = W96.T
    brow = bc.reshape(1, _C)
    bcol = bc.reshape(_C, 1)

    resp = _resp(colA, WfT, brow)
    resp_blk = _block_resp(resp)
    col128 = colA.reshape(_B * _N // 4, 128)
    sel = _select_gather(resp_blk, col128)
    po = sel.reshape(_B, _K, 128)
    pT = po[:, :, :_KP].transpose(0, 2, 1)
    rA = po[:, :, 32].reshape(_B, 1, _K)
    cA = po[:, :, 33].reshape(_B, 1, _K)
    dr, dc, mv = _dist(colB, pT, rA, cA, WfT, W96, brow, bcol)
    return jnp.stack([dr[:, 0, :], dc[:, 0, :], mv[:, 0, :]], axis=-1)


# speedup vs baseline: 1.3341x; 1.3341x over previous
import functools
import jax
import jax.numpy as jnp
from jax import lax
from jax.experimental import pallas as pl
from jax.experimental.pallas import tpu as pltpu
from jax.experimental.pallas import tpu_sc as plsc

_B = 4
_C = 96
_H = 224
_W = 224
_N = _H * _W
_P = 8
_BLK = 28
_K = 64
_KP = 32
_TN = 6272
_NT = _N // _TN
_BPAD = 896
_NSUB = 32
_TPS = (_B * _K) // _NSUB


def _im2col(x):
    xp = jnp.pad(x, ((0, 0), (0, 0), (1, 1), (1, 1)))
    sl = [xp[:, :, dh:dh + _H, dw:dw + _W] for dh in range(3) for dw in range(3)]
    col = jnp.stack(sl, axis=-1)
    col = col.transpose(0, 2, 3, 1, 4).reshape(_B, _N, 27)
    return jnp.pad(col, ((0, 0), (0, 0), (0, _KP - 27)))


def _resp_body(col_ref, wt_ref, b_ref, out_ref):
    x = col_ref[0]
    f = jnp.dot(x, wt_ref[...], preferred_element_type=jnp.float32)
    f = jnp.maximum(f + b_ref[...], 0.0)
    out_ref[0] = lax.dot_general(
        jnp.ones((1, _C), jnp.float32), f,
        dimension_numbers=(((1,), (1,)), ((), ())),
        preferred_element_type=jnp.float32)


def _resp(colA, WfT, brow):
    return pl.pallas_call(
        _resp_body,
        grid=(_B * _NT,),
        in_specs=[
            pl.BlockSpec((1, _TN, _KP), lambda g: (g // _NT, g % _NT, 0)),
            pl.BlockSpec((_KP, _C), lambda g: (0, 0)),
            pl.BlockSpec((1, _C), lambda g: (0, 0)),
        ],
        out_specs=pl.BlockSpec((1, 1, _TN), lambda g: (g, 0, 0)),
        out_shape=jax.ShapeDtypeStruct((_B * _NT, 1, _TN), jnp.float32),
    )(colA, WfT, brow)


def _lane_gather(x, idx):
    dn = lax.GatherDimensionNumbers(
        offset_dims=(), collapsed_slice_dims=(0,), start_index_map=(0,))
    return lax.gather(x, idx[:, None], dn, slice_sizes=(1,),
                      mode=lax.GatherScatterMode.PROMISE_IN_BOUNDS)


def _sel_body(resp_hbm, col_hbm, out_hbm, band, rows, outv, sem):
    wid = lax.axis_index("s") * 2 + lax.axis_index("c")
    lanes = lax.iota(jnp.int32, 16)
    b = wid >> 3
    p = wid & 7
    pltpu.sync_copy(resp_hbm.at[wid, 0], band)
    gidx_l = []
    rc_l = []
    for q in range(_P):
        def rowbody(rr, carry, q=q):
            bv, bi = carry
            base = rr * _W + q * _BLK
            v0 = band[pl.ds(base, 16)]
            v1 = band[pl.ds(base + 12, 16)]
            i0 = rr * _BLK + lanes
            i1 = i0 + 12
            c0 = (v0 > bv) | ((v0 == bv) & (i0 < bi))
            bv = jnp.where(c0, v0, bv)
            bi = jnp.where(c0, i0, bi)
            c1 = (v1 > bv) | ((v1 == bv) & (i1 < bi))
            bv = jnp.where(c1, v1, bv)
            bi = jnp.where(c1, i1, bi)
            return bv, bi

        bv, bi = lax.fori_loop(0, _BLK, rowbody,
                               (jnp.full((16,), -jnp.inf, jnp.float32),
                                jnp.full((16,), 1 << 20, jnp.int32)))
        cv, ci = bv, bi
        for s in (8, 4, 2, 1):
            perm = (lanes + s) & 15
            ov = _lane_gather(cv, perm)
            oi = _lane_gather(ci, perm)
            take = (ov > cv) | ((ov == cv) & (oi < ci))
            cv = jnp.where(take, ov, cv)
            ci = jnp.where(take, oi, ci)
        qq = ((ci.astype(jnp.float32) + 0.5) * (1.0 / _BLK)).astype(jnp.int32)
        r = ci - qq * _BLK
        row = p * _BLK + qq
        col = q * _BLK + r
        gidx_l.append(b * _N + row * _W + col)
        rc_l.append(jnp.where(lanes == 0, row.astype(jnp.float32),
                              jnp.where(lanes == 1, col.astype(jnp.float32),
                                        0.0)))
    copies = [pltpu.make_async_copy(col_hbm.at[gidx_l[q][0]], rows.at[q], sem)
              for q in range(_P)]
    for c in copies:
        c.start()
    for c in copies:
        c.wait()
    zero = jnp.zeros((16,), jnp.float32)
    for q in range(_P):
        outv[q, pl.ds(0, 16)] = rows[q, pl.ds(0, 16)]
        outv[q, pl.ds(16, 16)] = rows[q, pl.ds(16, 16)]
        outv[q, pl.ds(32, 16)] = rc_l[q]
        for c in range(3, 8):
            outv[q, pl.ds(c * 16, 16)] = zero
    pltpu.sync_copy(outv, out_hbm.at[pl.ds(wid * _TPS, _TPS)])


def _select_gather(resp3, col_flat):
    mesh = plsc.VectorSubcoreMesh(core_axis_name="c", subcore_axis_name="s")
    f = pl.kernel(
        _sel_body,
        mesh=mesh,
        out_type=jax.ShapeDtypeStruct((_B * _K, 128), jnp.float32),
        scratch_types=[
            pltpu.VMEM((_TN,), jnp.float32),
            pltpu.VMEM((_TPS, _KP), jnp.float32),
            pltpu.VMEM((_TPS, 128), jnp.float32),
            pltpu.SemaphoreType.DMA,
        ],
        compiler_params=pltpu.CompilerParams(
            use_tc_tiling_on_sc=True, needs_layout_passes=False),
    )
    return f(resp3, col_flat)


def _dist_body(col_ref, pT_ref, rA_ref, cA_ref, wt_ref, w96_ref, brow_ref,
               bcol_ref, dr_ref, dc_ref, mv_ref, descT, nA, rmin, ridx):
    nt = pl.program_id(1)

    @pl.when(nt == 0)
    def _():
        d = jnp.dot(w96_ref[...], pT_ref[0], preferred_element_type=jnp.float32)
        d = jnp.maximum(d + bcol_ref[...], 0.0)
        descT[...] = d
        nA[...] = jnp.sum(d * d, axis=0, keepdims=True)
        rmin[...] = jnp.full((1, _K), jnp.inf, jnp.float32)
        ridx[...] = jnp.zeros((1, _K), jnp.int32)

    x = col_ref[0]
    f = jnp.dot(x, wt_ref[...], preferred_element_type=jnp.float32)
    f = jnp.maximum(f + brow_ref[...], 0.0)
    dots = jnp.dot(f, descT[...], preferred_element_type=jnp.float32)
    nb = jnp.sum(f * f, axis=1, keepdims=True)
    dist = nb - 2.0 * dots
    tmin = jnp.min(dist, axis=0, keepdims=True)
    ii = lax.broadcasted_iota(jnp.int32, (_TN, _K), 0)
    targ = jnp.min(jnp.where(dist == tmin, ii, jnp.int32(_TN)),
                   axis=0, keepdims=True)
    better = tmin < rmin[...]
    ridx[...] = jnp.where(better, targ + nt * _TN, ridx[...])
    rmin[...] = jnp.where(better, tmin, rmin[...])

    @pl.when(nt == _NT - 1)
    def _():
        idx = ridx[...]
        rB = (idx // _W).astype(jnp.float32)
        cB = (idx % _W).astype(jnp.float32)
        dr_ref[0] = rA_ref[0] - rB
        dc_ref[0] = cA_ref[0] - cB
        mv_ref[0] = rmin[...] + nA[...]


def _dist(colB, pT, rA, cA, WfT, W96, brow, bcol):
    out3 = [jax.ShapeDtypeStruct((_B, 1, _K), jnp.float32)] * 3
    return pl.pallas_call(
        _dist_body,
        grid=(_B, _NT),
        in_specs=[
            pl.BlockSpec((1, _TN, _KP), lambda b, n: (b, n, 0)),
            pl.BlockSpec((1, _KP, _K), lambda b, n: (b, 0, 0)),
            pl.BlockSpec((1, 1, _K), lambda b, n: (b, 0, 0)),
            pl.BlockSpec((1, 1, _K), lambda b, n: (b, 0, 0)),
            pl.BlockSpec((_KP, _C), lambda b, n: (0, 0)),
            pl.BlockSpec((_C, _KP), lambda b, n: (0, 0)),
            pl.BlockSpec((1, _C), lambda b, n: (0, 0)),
            pl.BlockSpec((_C, 1), lambda b, n: (0, 0)),
        ],
        out_specs=[pl.BlockSpec((1, 1, _K), lambda b, n: (b, 0, 0))] * 3,
        out_shape=out3,
        scratch_shapes=[
            pltpu.VMEM((_C, _K), jnp.float32),
            pltpu.VMEM((1, _K), jnp.float32),
            pltpu.VMEM((1, _K), jnp.float32),
            pltpu.VMEM((1, _K), jnp.int32),
        ],
        compiler_params=pltpu.CompilerParams(
            dimension_semantics=("arbitrary", "arbitrary")),
    )(colB, pT, rA, cA, WfT, W96, brow, bcol)


@jax.jit
def kernel(xA, xB, Wc, bc):
    colA = _im2col(xA)
    colB = _im2col(xB)
    Wf = Wc.reshape(_C, 27)
    W96 = jnp.pad(Wf, ((0, 0), (0, _KP - 27)))
    WfT = W96.T
    brow = bc.reshape(1, _C)
    bcol = bc.reshape(_C, 1)

    resp3 = _resp(colA, WfT, brow)
    sel = _select_gather(resp3, colA.reshape(_B * _N, _KP))
    po = sel.reshape(_B, _K, 128)
    pT = po[:, :, :_KP].transpose(0, 2, 1)
    rA = po[:, :, 32].reshape(_B, 1, _K)
    cA = po[:, :, 33].reshape(_B, 1, _K)
    dr, dc, mv = _dist(colB, pT, rA, cA, WfT, W96, brow, bcol)
    return jnp.stack([dr[:, 0, :], dc[:, 0, :], mv[:, 0, :]], axis=-1)
